# baseline (speedup 1.0000x reference)
import functools
import math

import jax
import jax.numpy as jnp
from jax import lax
from jax.experimental import pallas as pl
from jax.experimental.pallas import tpu as pltpu
from jax.experimental.pallas import tpu_sc as plsc

_VOCAB = 100000
_EMB = 64
_B = 4096
_L = 200
_SCALE = math.sqrt(_EMB)

_NC = 2
_NS = 16
_NW = _NC * _NS

_B_TOTAL = _B * _L
_B_PER_W = _B_TOTAL // _NW
_CHUNK = 400
_N_CHUNKS = _B_PER_W // _CHUNK
_NBUF = 4


_RCHUNK = 400
_N_RCHUNKS_TOT = _VOCAB // _RCHUNK


@functools.partial(
    pl.kernel,
    out_type=jax.ShapeDtypeStruct((_VOCAB * _EMB,), jnp.float32),
    mesh=plsc.VectorSubcoreMesh(core_axis_name="c", subcore_axis_name="s"),
    scratch_types=[
        pltpu.VMEM((_RCHUNK, _EMB), jnp.float32),
        pltpu.VMEM((_RCHUNK * _EMB,), jnp.float32),
    ],
    compiler_params=pltpu.CompilerParams(use_tc_tiling_on_sc=True),
)
def _sc_scale(table_hbm, out_hbm, in_v, out_v):
    wid = lax.axis_index("s") * _NC + lax.axis_index("c")
    nk = (_N_RCHUNKS_TOT - wid + _NW - 1) // _NW

    @pl.loop(0, nk)
    def _rchunk_loop(k):
        row = (wid + k * _NW) * _RCHUNK
        pltpu.sync_copy(table_hbm.at[pl.ds(row, _RCHUNK)], in_v)

        @pl.loop(0, _RCHUNK)
        def _row_loop(i):
            for j in range(_EMB // 16):
                v = in_v[i, pl.ds(j * 16, 16)]
                out_v[pl.ds(i * _EMB + j * 16, 16)] = v * _SCALE

        pltpu.sync_copy(
            out_v, out_hbm.at[pl.ds(row * _EMB, _RCHUNK * _EMB)])


@functools.partial(
    pl.kernel,
    out_type=jax.ShapeDtypeStruct((_B_TOTAL, _EMB), jnp.float32),
    mesh=plsc.VectorSubcoreMesh(core_axis_name="c", subcore_axis_name="s"),
    scratch_types=[
        pltpu.VMEM((_B_PER_W,), jnp.int32),
        pltpu.VMEM((_NBUF, _CHUNK, _EMB), jnp.float32),
        pltpu.SemaphoreType.DMA((_NBUF,)),
        pltpu.SemaphoreType.DMA((_NBUF,)),
    ],
    compiler_params=pltpu.CompilerParams(use_tc_tiling_on_sc=False),
)
def _sc_gather(table_hbm, tok_hbm, out_hbm, idx_v, rows_v, sem_g, sem_o):
    wid = lax.axis_index("s") * _NC + lax.axis_index("c")
    base = wid * _B_PER_W

    pltpu.sync_copy(tok_hbm.at[pl.ds(base, _B_PER_W)], idx_v)

    def start_gather(b, g):
        pltpu.async_copy(
            table_hbm.at[idx_v.at[pl.ds(g * _CHUNK, _CHUNK)]],
            rows_v.at[b], sem_g.at[b])

    def wait_gather(b):
        pltpu.make_async_copy(
            table_hbm.at[idx_v.at[pl.ds(0, _CHUNK)]],
            rows_v.at[b], sem_g.at[b]).wait()

    def start_out(b, g):
        pltpu.async_copy(
            rows_v.at[b], out_hbm.at[pl.ds(base + g * _CHUNK, _CHUNK)],
            sem_o.at[b])

    def wait_out(b):
        pltpu.make_async_copy(
            rows_v.at[b], out_hbm.at[pl.ds(base, _CHUNK)],
            sem_o.at[b]).wait()

    for b in range(_NBUF):
        start_gather(b, b)

    @pl.loop(0, _N_CHUNKS // _NBUF - 1)
    def _group_loop(gi):
        i = gi * _NBUF
        for b in range(_NBUF):
            wait_gather(b)
            start_out(b, i + b)
        for b in range(_NBUF):
            wait_out(b)
            start_gather(b, i + _NBUF + b)

    i_last = _N_CHUNKS - _NBUF
    for b in range(_NBUF):
        wait_gather(b)
        start_out(b, i_last + b)
    for b in range(_NBUF):
        wait_out(b)


def kernel(tokens, word_vectors):
    scaled = _sc_scale(word_vectors).reshape(_VOCAB, _EMB)
    flat = tokens.reshape(_B_TOTAL)
    out = _sc_gather(scaled, flat)
    return out.reshape(_B, _L, _EMB)


# speedup vs baseline: 1.6492x; 1.6492x over previous
import functools
import math

import jax
import jax.numpy as jnp
from jax import lax
from jax.experimental import pallas as pl
from jax.experimental.pallas import tpu as pltpu
from jax.experimental.pallas import tpu_sc as plsc

_VOCAB = 100000
_EMB = 64
_B = 4096
_L = 200
_SCALE = math.sqrt(_EMB)

_NC = 2
_NS = 16
_NW = _NC * _NS

_B_TOTAL = _B * _L
_B_PER_W = _B_TOTAL // _NW
_CHUNK = 400
_N_CHUNKS = _B_PER_W // _CHUNK
_NBUF = 4


_RCHUNK = 400
_N_RCHUNKS_TOT = _VOCAB // _RCHUNK


@functools.partial(
    pl.kernel,
    out_type=jax.ShapeDtypeStruct((_VOCAB * _EMB,), jnp.float32),
    mesh=plsc.VectorSubcoreMesh(core_axis_name="c", subcore_axis_name="s"),
    scratch_types=[
        pltpu.VMEM((_RCHUNK, _EMB), jnp.float32),
        pltpu.VMEM((_RCHUNK * _EMB,), jnp.float32),
    ],
    compiler_params=pltpu.CompilerParams(use_tc_tiling_on_sc=True),
)
def _sc_scale(table_hbm, out_hbm, in_v, out_v):
    wid = lax.axis_index("s") * _NC + lax.axis_index("c")
    nk = (_N_RCHUNKS_TOT - wid + _NW - 1) // _NW

    @pl.loop(0, nk)
    def _rchunk_loop(k):
        row = (wid + k * _NW) * _RCHUNK
        pltpu.sync_copy(table_hbm.at[pl.ds(row, _RCHUNK)], in_v)

        @pl.loop(0, _RCHUNK)
        def _row_loop(i):
            for j in range(_EMB // 16):
                v = in_v[i, pl.ds(j * 16, 16)]
                out_v[pl.ds(i * _EMB + j * 16, 16)] = v * _SCALE

        pltpu.sync_copy(
            out_v, out_hbm.at[pl.ds(row * _EMB, _RCHUNK * _EMB)])


@functools.partial(
    pl.kernel,
    out_type=jax.ShapeDtypeStruct((_B_TOTAL, 128), jnp.float32),
    mesh=plsc.VectorSubcoreMesh(core_axis_name="c", subcore_axis_name="s"),
    scratch_types=[
        pltpu.VMEM((_B_PER_W,), jnp.int32),
        pltpu.VMEM((_NBUF, _CHUNK, _EMB), jnp.float32),
        pltpu.SemaphoreType.DMA((_NBUF,)),
        pltpu.SemaphoreType.DMA((_NBUF,)),
    ],
    compiler_params=pltpu.CompilerParams(use_tc_tiling_on_sc=False),
)
def _sc_gather(table_hbm, tok_hbm, out_hbm, idx_v, rows_v, sem_g, sem_o):
    wid = lax.axis_index("s") * _NC + lax.axis_index("c")
    base = wid * _B_PER_W

    pltpu.sync_copy(tok_hbm.at[pl.ds(base, _B_PER_W)], idx_v)

    def start_gather(b, g):
        pltpu.async_copy(
            table_hbm.at[idx_v.at[pl.ds(g * _CHUNK, _CHUNK)]],
            rows_v.at[b], sem_g.at[b])

    def wait_gather(b):
        pltpu.make_async_copy(
            table_hbm.at[idx_v.at[pl.ds(0, _CHUNK)]],
            rows_v.at[b], sem_g.at[b]).wait()

    def start_out(b, g):
        pltpu.async_copy(
            rows_v.at[b],
            out_hbm.at[pl.ds(base + g * _CHUNK, _CHUNK), pl.ds(0, _EMB)],
            sem_o.at[b])

    def wait_out(b):
        pltpu.make_async_copy(
            rows_v.at[b],
            out_hbm.at[pl.ds(base, _CHUNK), pl.ds(0, _EMB)],
            sem_o.at[b]).wait()

    for b in range(_NBUF):
        start_gather(b, b)

    @pl.loop(0, _N_CHUNKS // _NBUF - 1)
    def _group_loop(gi):
        i = gi * _NBUF
        for b in range(_NBUF):
            wait_gather(b)
            start_out(b, i + b)
        for b in range(_NBUF):
            wait_out(b)
            start_gather(b, i + _NBUF + b)

    i_last = _N_CHUNKS - _NBUF
    for b in range(_NBUF):
        wait_gather(b)
        start_out(b, i_last + b)
    for b in range(_NBUF):
        wait_out(b)


def kernel(tokens, word_vectors):
    scaled = _sc_scale(word_vectors).reshape(_VOCAB, _EMB)
    flat = tokens.reshape(_B_TOTAL)
    out = _sc_gather(scaled, flat)
    return out[:, :_EMB].reshape(_B, _L, _EMB)
